# baseline (speedup 1.0000x reference)
import jax
import jax.numpy as jnp
from jax import lax
from jax.experimental import pallas as pl

B = 1024
CSV = 512
ED = 256
FD = ED * 3
E = 8
NC = 12
EPS = 1e-5
f32 = jnp.float32
bf16 = jnp.bfloat16


def _csv_kernel(x_ref, w1_ref, b1_ref, g1_ref, be1_ref,
                w2_ref, b2_ref, g2_ref, be2_ref, o_ref):
    h = jnp.dot(x_ref[...], w1_ref[...], preferred_element_type=f32) + b1_ref[...]
    m = jnp.mean(h, axis=0, keepdims=True)
    v = jnp.mean((h - m) ** 2, axis=0, keepdims=True)
    h = jnp.maximum((h - m) / jnp.sqrt(v + EPS) * g1_ref[...] + be1_ref[...], 0.0)
    h2 = jnp.dot(h.astype(bf16), w2_ref[...], preferred_element_type=f32) + b2_ref[...]
    m2 = jnp.mean(h2, axis=0, keepdims=True)
    v2 = jnp.mean((h2 - m2) ** 2, axis=0, keepdims=True)
    o_ref[...] = jnp.maximum((h2 - m2) / jnp.sqrt(v2 + EPS) * g2_ref[...] + be2_ref[...], 0.0)


def _fc_kernel(x_ref, fw_ref, fb_ref, o_ref):
    o_ref[...] = (jnp.dot(x_ref[...], fw_ref[...], preferred_element_type=f32)
                  + fb_ref[...])


def _head_kernel(hcsv_ref, fc_ref, g3_ref, be3_ref, mask_ref,
                 iw1_ref, ib1_ref, iw2_ref, ib2_ref, gw_ref, gb_ref,
                 x16_ref, lg_ref):
    def img_bn(fc):
        m = jnp.mean(fc, axis=0, keepdims=True)
        v = jnp.mean((fc - m) ** 2, axis=0, keepdims=True)
        return jnp.maximum((fc - m) / jnp.sqrt(v + EPS) * g3_ref[...] + be3_ref[...], 0.0)

    h1 = img_bn(fc_ref[0:B])
    h2 = img_bn(fc_ref[B:2 * B])
    m0 = mask_ref[:, 0:1]
    m1 = mask_ref[:, 1:2]
    m2 = mask_ref[:, 2:3]
    hc = hcsv_ref[...] * m0
    h1 = h1 * m1
    h2 = h2 * m2
    fused = jnp.concatenate([hc, h1, h2], axis=1)
    t = jnp.maximum(jnp.dot(fused.astype(bf16), iw1_ref[...], preferred_element_type=f32)
                    + ib1_ref[...], 0.0)
    raw = jnp.dot(t.astype(bf16), iw2_ref[...], preferred_element_type=f32) + ib2_ref[...]
    f_csv = hc * m0 + raw[:, 0:ED] * (1 - m0)
    f_i1 = h1 * m1 + raw[:, ED:2 * ED] * (1 - m1)
    f_i2 = h2 * m2 + raw[:, 2 * ED:3 * ED] * (1 - m2)
    xf = jnp.concatenate([f_csv, f_i1, f_i2], axis=1)
    x16 = xf.astype(bf16)
    x16_ref[...] = x16
    lg_ref[...] = jnp.dot(x16, gw_ref[...], preferred_element_type=f32) + gb_ref[...]


def _expert_kernel(x16_ref, lg_ref, ew1_ref, eb1_ref, ew2_ref, eb2_ref, o_ref):
    lg = lg_ref[:, 0:E]
    iota = lax.broadcasted_iota(jnp.int32, (B, E), 1)
    m1 = jnp.max(lg, axis=1, keepdims=True)
    i1 = jnp.min(jnp.where(lg == m1, iota, E + 1), axis=1, keepdims=True)
    sel1 = iota == i1
    lg2 = jnp.where(sel1, -1e30, lg)
    m2 = jnp.max(lg2, axis=1, keepdims=True)
    i2 = jnp.min(jnp.where(lg2 == m2, iota, E + 1), axis=1, keepdims=True)
    sel2 = iota == i2
    conf = 1.0 / (1.0 + jnp.exp(-lg))
    w = jnp.where(sel1 | sel2, conf, 0.0)

    x = x16_ref[...]
    h_all = jnp.maximum(jnp.dot(x, ew1_ref[...], preferred_element_type=f32)
                        + eb1_ref[...], 0.0)
    final = jnp.zeros((B, NC), f32)
    for e in range(E):
        he = h_all[:, e * 128:(e + 1) * 128].astype(bf16)
        eo = jnp.dot(he, ew2_ref[e], preferred_element_type=f32) + eb2_ref[e:e + 1, :]
        final = final + w[:, e:e + 1] * eo
    o_ref[...] = final


def _b16(x):
    return x.astype(bf16).astype(f32)


def _convx(x, W, b):
    y = lax.conv_general_dilated(_b16(x), _b16(W), (1, 1), 'SAME',
                                 dimension_numbers=('NCHW', 'OIHW', 'NCHW'),
                                 precision=lax.Precision.HIGHEST)
    return y + b.reshape(1, -1, 1, 1)


def _mp(x):
    return lax.reduce_window(x, -jnp.inf, lax.max, (1, 1, 2, 2), (1, 1, 2, 2), 'VALID')


def _bn2d(x, g, b):
    m = jnp.mean(x, axis=(0, 2, 3), keepdims=True)
    v = jnp.var(x, axis=(0, 2, 3), keepdims=True)
    return (x - m) / jnp.sqrt(v + EPS) * g.reshape(1, -1, 1, 1) + b.reshape(1, -1, 1, 1)


def _img_convs(x, p):
    xn = jnp.transpose(x, (0, 3, 1, 2))
    h = _mp(jax.nn.relu(_bn2d(_convx(xn, p['cW1'], p['cb1']), p['g1'], p['be1'])))
    h = _mp(jax.nn.relu(_bn2d(_convx(h, p['cW2'], p['cb2']), p['g2'], p['be2'])))
    return h.reshape(h.shape[0], -1)


def kernel(x_csv, x_img1, x_img2, mask, params):
    pc = params['csv']
    pi = params['img']
    pm = params['imp']
    pg = params['gate']
    pe = params['experts']

    h_csv = pl.pallas_call(
        _csv_kernel,
        out_shape=jax.ShapeDtypeStruct((B, ED), f32),
    )(x_csv.astype(bf16), pc['W1'].T.astype(bf16), pc['b1'].reshape(1, -1),
      pc['g1'].reshape(1, -1), pc['be1'].reshape(1, -1),
      pc['W2'].T.astype(bf16), pc['b2'].reshape(1, -1),
      pc['g2'].reshape(1, -1), pc['be2'].reshape(1, -1))

    hflat = jnp.concatenate([_img_convs(x_img1, pi), _img_convs(x_img2, pi)], axis=0)
    fc_raw = pl.pallas_call(
        _fc_kernel,
        out_shape=jax.ShapeDtypeStruct((2 * B, ED), f32),
    )(hflat.astype(bf16), pi['fW'].T.astype(bf16), pi['fb'].reshape(1, ED))

    maskp = jnp.concatenate([mask, jnp.zeros((B, 5), f32)], axis=1)
    gwp = jnp.concatenate([pg['W'].T, jnp.zeros((FD, 128 - E), f32)], axis=1).astype(bf16)
    gbp = jnp.concatenate([pg['b'], jnp.zeros((128 - E,), f32)]).reshape(1, 128)
    x16, logits = pl.pallas_call(
        _head_kernel,
        out_shape=[
            jax.ShapeDtypeStruct((B, FD), bf16),
            jax.ShapeDtypeStruct((B, 128), f32),
        ],
    )(h_csv, fc_raw, pi['g3'].reshape(1, ED), pi['be3'].reshape(1, ED), maskp,
      pm['W1'].T.astype(bf16), pm['b1'].reshape(1, -1),
      pm['W2'].T.astype(bf16), pm['b2'].reshape(1, -1), gwp, gbp)

    ew1 = pe['W1'].transpose(2, 0, 1).reshape(FD, E * 128).astype(bf16)
    eb1 = pe['b1'].reshape(1, E * 128)
    ew2 = pe['W2'].transpose(0, 2, 1).astype(bf16)
    final = pl.pallas_call(
        _expert_kernel,
        out_shape=jax.ShapeDtypeStruct((B, NC), f32),
    )(x16, logits, ew1, eb1, ew2, pe['b2'])
    return final
